# baseline (speedup 1.0000x reference)
import functools

import jax
import jax.numpy as jnp
from jax import lax
from jax.experimental import pallas as pl
from jax.experimental.pallas import tpu as pltpu
from jax.experimental.pallas import tpu_sc as plsc

_NC, _NS, _L = 2, 16, 16
_NW = _NC * _NS
_KT = 32


def _sc_call(x_flat, t_flat, B, T, C):
    span = T // _NW
    nchunks = span // _KT
    chunk = _KT * C

    mesh = plsc.VectorSubcoreMesh(core_axis_name="c", subcore_axis_name="s")

    @functools.partial(
        pl.kernel,
        mesh=mesh,
        out_type=jax.ShapeDtypeStruct((B * T * C,), jnp.float32),
        scratch_types=[
            pltpu.VMEM((chunk,), jnp.float32),
            pltpu.VMEM((chunk,), jnp.float32),
        ],
    )
    def k(x_hbm, t_hbm, out_hbm, tbuf, xbuf):
        wid = lax.axis_index("s") * _NC + lax.axis_index("c")
        t0 = wid * span
        for c in range(nchunks):
            tok = t0 + c * _KT
            pltpu.sync_copy(t_hbm.at[pl.ds(tok * C, chunk)], tbuf)
            for b in range(B):
                off = (b * T + tok) * C
                pltpu.sync_copy(x_hbm.at[pl.ds(off, chunk)], xbuf)

                def body(i, carry):
                    s = i * _L
                    xbuf[pl.ds(s, _L)] = xbuf[pl.ds(s, _L)] + tbuf[pl.ds(s, _L)]
                    return carry

                lax.fori_loop(0, chunk // _L, body, 0)
                pltpu.sync_copy(xbuf, out_hbm.at[pl.ds(off, chunk)])

    return k(x_flat, t_flat)


def kernel(x, pos_table):
    B, T, C = x.shape
    out = _sc_call(x.reshape(-1), pos_table.reshape(-1), B, T, C)
    return out.reshape(B, T, C)
